# baseline (speedup 1.0000x reference)
import functools

import jax
import jax.numpy as jnp
from jax import lax
from jax.experimental import pallas as pl
from jax.experimental.pallas import tpu as pltpu
from jax.experimental.pallas import tpu_sc as plsc

B = 16
NV = 6890
H = 512
W = 512
C = 3
P = H * W

_info = plsc.get_sparse_core_info()
NC = _info.num_cores
NS = _info.num_subcores
L = _info.num_lanes
NW = NC * NS
PPW = P // NW
HPW = H // NW
NVP = ((NV * C + 15) // 16) * 16
NCH = 2
CHPX = PPW // NCH


def _sc_body(
    verts_hbm, idx_hbm, bary_hbm, out_hbm,
    idx_v, bary_v, vbuf, obuf0, obuf1, osem0, osem1,
):
    obuf = (obuf0, obuf1)
    osem = (osem0, osem1)
    wid = lax.axis_index("s") * NC + lax.axis_index("c")
    base_px = wid * PPW
    row0 = wid * HPW

    for k in range(C):
        pltpu.sync_copy(
            idx_hbm.at[pl.ds(k * P + base_px, PPW)],
            idx_v.at[pl.ds(k * PPW, PPW)],
        )
        pltpu.sync_copy(
            bary_hbm.at[pl.ds(k * P + base_px, PPW)],
            bary_v.at[pl.ds(k * PPW, PPW)],
        )

    iota = lax.iota(jnp.int32, L)

    def make_px_body(oslot, chunk):
        def px_body(i, _):
            s = chunk * CHPX + i * L
            i0 = idx_v[pl.ds(s, L)]
            i1 = idx_v[pl.ds(PPW + s, L)]
            i2 = idx_v[pl.ds(2 * PPW + s, L)]
            b0 = bary_v[pl.ds(s, L)]
            b1 = bary_v[pl.ds(PPW + s, L)]
            b2 = bary_v[pl.ds(2 * PPW + s, L)]
            lr = i // (W // L)
            lrv = jnp.full((L,), lr, jnp.int32)
            cv = (i % (W // L)) * (L * C) + iota * C
            for c in range(C):
                g0 = plsc.load_gather(vbuf, [i0 + c])
                g1 = plsc.load_gather(vbuf, [i1 + c])
                g2 = plsc.load_gather(vbuf, [i2 + c])
                acc = b0 * g0 + b1 * g1 + b2 * g2
                plsc.store_scatter(obuf[oslot], [lrv, cv + c], acc)
            return _

        return px_body

    ocopies = [None, None]
    for b in range(B):
        pltpu.sync_copy(verts_hbm.at[b], vbuf)
        for chunk in range(NCH):
            oslot = (b * NCH + chunk) % 2
            if ocopies[oslot] is not None:
                ocopies[oslot].wait()
            lax.fori_loop(
                0, CHPX // L, make_px_body(oslot, chunk), 0, unroll=4
            )
            ocopies[oslot] = pltpu.async_copy(
                obuf[oslot],
                out_hbm.at[b, pl.ds(row0 + chunk * (HPW // NCH), HPW // NCH)],
                osem[oslot],
            )
    for oc in ocopies:
        if oc is not None:
            oc.wait()


@functools.partial(jax.jit, static_argnames=())
def kernel(verts, bary_weights, v_index):
    idx3 = (v_index.reshape(P, C).astype(jnp.int32) * 3).T.reshape(C * P)
    bary = bary_weights.reshape(P, C).T.reshape(C * P)
    verts_flat = jnp.pad(
        verts.reshape(B, NV * C), ((0, 0), (0, NVP - NV * C))
    )

    sc = pl.kernel(
        _sc_body,
        mesh=plsc.VectorSubcoreMesh(core_axis_name="c", subcore_axis_name="s"),
        out_type=jax.ShapeDtypeStruct((B, H, W * C), jnp.float32),
        scratch_types=[
            pltpu.VMEM((C * PPW,), jnp.int32),
            pltpu.VMEM((C * PPW,), jnp.float32),
            pltpu.VMEM((NVP,), jnp.float32),
            pltpu.VMEM((HPW // NCH, W * C), jnp.float32),
            pltpu.VMEM((HPW // NCH, W * C), jnp.float32),
            pltpu.SemaphoreType.DMA,
            pltpu.SemaphoreType.DMA,
        ],
        compiler_params=pltpu.CompilerParams(needs_layout_passes=False),
    )
    return sc(verts_flat, idx3, bary).reshape(B, H, W, C)


# speedup vs baseline: 1.4663x; 1.4663x over previous
import functools

import jax
import jax.numpy as jnp
from jax import lax
from jax.experimental import pallas as pl
from jax.experimental.pallas import tpu as pltpu
from jax.experimental.pallas import tpu_sc as plsc

B = 16
NV = 6890
H = 512
W = 512
C = 3
P = H * W

_info = plsc.get_sparse_core_info()
NC = _info.num_cores
NS = _info.num_subcores
L = _info.num_lanes
NW = NC * NS
PPW = P // NW
HPW = H // NW
NVP = ((NV * C + 15) // 16) * 16
NCH = 2
CHPX = PPW // NCH


def _sc_body(
    verts_hbm, idx_hbm, bary_hbm, out_hbm,
    idx_v, bary_v, vbuf, obuf0, obuf1, osem0, osem1,
):
    obuf = (obuf0, obuf1)
    osem = (osem0, osem1)
    wid = lax.axis_index("s") * NC + lax.axis_index("c")
    base_px = wid * PPW
    row0 = wid * HPW

    pltpu.sync_copy(idx_hbm.at[:, pl.ds(base_px, PPW)], idx_v)
    pltpu.sync_copy(bary_hbm.at[:, pl.ds(base_px, PPW)], bary_v)

    iota = lax.iota(jnp.int32, L)

    def make_px_body(oslot, chunk):
        def px_body(i):
            s = chunk * CHPX + i * L
            i0 = idx_v[0, pl.ds(s, L)]
            i1 = idx_v[1, pl.ds(s, L)]
            i2 = idx_v[2, pl.ds(s, L)]
            b0 = bary_v[0, pl.ds(s, L)]
            b1 = bary_v[1, pl.ds(s, L)]
            b2 = bary_v[2, pl.ds(s, L)]
            lr = i // (W // L)
            lrv = jnp.full((L,), lr, jnp.int32)
            cv = (i % (W // L)) * (L * C) + iota * C
            for c in range(C):
                g0 = plsc.load_gather(vbuf, [i0 + c])
                g1 = plsc.load_gather(vbuf, [i1 + c])
                g2 = plsc.load_gather(vbuf, [i2 + c])
                acc = b0 * g0 + b1 * g1 + b2 * g2
                plsc.store_scatter(obuf[oslot], [lrv, cv + c], acc)

        return px_body

    ocopies = [None, None]
    for b in range(B):
        pltpu.sync_copy(verts_hbm.at[b], vbuf)
        for chunk in range(NCH):
            oslot = (b * NCH + chunk) % 2
            if ocopies[oslot] is not None:
                ocopies[oslot].wait()
            plsc.parallel_loop(0, CHPX // L, unroll=4)(
                make_px_body(oslot, chunk)
            )
            ocopies[oslot] = pltpu.async_copy(
                obuf[oslot],
                out_hbm.at[b, pl.ds(row0 + chunk * (HPW // NCH), HPW // NCH)],
                osem[oslot],
            )
    for oc in ocopies:
        if oc is not None:
            oc.wait()


@functools.partial(jax.jit, static_argnames=())
def kernel(verts, bary_weights, v_index):
    idx3 = (v_index.reshape(P, C).astype(jnp.int32) * 3).T
    bary = bary_weights.reshape(P, C).T
    verts_flat = jnp.pad(
        verts.reshape(B, NV * C), ((0, 0), (0, NVP - NV * C))
    )

    sc = pl.kernel(
        _sc_body,
        mesh=plsc.VectorSubcoreMesh(core_axis_name="c", subcore_axis_name="s"),
        out_type=jax.ShapeDtypeStruct((B, H, W * C), jnp.float32),
        scratch_types=[
            pltpu.VMEM((C, PPW), jnp.int32),
            pltpu.VMEM((C, PPW), jnp.float32),
            pltpu.VMEM((NVP,), jnp.float32),
            pltpu.VMEM((HPW // NCH, W * C), jnp.float32),
            pltpu.VMEM((HPW // NCH, W * C), jnp.float32),
            pltpu.SemaphoreType.DMA,
            pltpu.SemaphoreType.DMA,
        ],
        compiler_params=pltpu.CompilerParams(needs_layout_passes=False),
    )
    return sc(verts_flat, idx3, bary).reshape(B, H, W, C)


# speedup vs baseline: 1.6079x; 1.0965x over previous
import functools

import jax
import jax.numpy as jnp
from jax import lax
from jax.experimental import pallas as pl
from jax.experimental.pallas import tpu as pltpu
from jax.experimental.pallas import tpu_sc as plsc

B = 16
NV = 6890
H = 512
W = 512
C = 3
P = H * W

_info = plsc.get_sparse_core_info()
NC = _info.num_cores
NS = _info.num_subcores
L = _info.num_lanes
NW = NC * NS
PPW = P // NW
HPW = H // NW
NVP = ((NV * C + 15) // 16) * 16
NCH = 4
CHPX = PPW // NCH


def _sc_body(
    verts_hbm, idx_hbm, bary_hbm, out_hbm,
    idx_v, bary_v, vbuf0, vbuf1, obuf0, obuf1,
    vsem0, vsem1, osem0, osem1,
):
    vbuf = (vbuf0, vbuf1)
    obuf = (obuf0, obuf1)
    vsem = (vsem0, vsem1)
    osem = (osem0, osem1)
    wid = lax.axis_index("s") * NC + lax.axis_index("c")
    base_px = wid * PPW
    row0 = wid * HPW

    pltpu.sync_copy(idx_hbm.at[:, pl.ds(base_px, PPW)], idx_v)
    pltpu.sync_copy(bary_hbm.at[:, pl.ds(base_px, PPW)], bary_v)

    iota = lax.iota(jnp.int32, L)

    def make_px_body(vslot, oslot, chunk):
        def px_body(i):
            s = chunk * CHPX + i * L
            i0 = idx_v[0, pl.ds(s, L)]
            i1 = idx_v[1, pl.ds(s, L)]
            i2 = idx_v[2, pl.ds(s, L)]
            b0 = bary_v[0, pl.ds(s, L)]
            b1 = bary_v[1, pl.ds(s, L)]
            b2 = bary_v[2, pl.ds(s, L)]
            lr = i // (W // L)
            lrv = jnp.full((L,), lr, jnp.int32)
            cv = (i % (W // L)) * (L * C) + iota * C
            for c in range(C):
                g0 = plsc.load_gather(vbuf[vslot], [i0 + c])
                g1 = plsc.load_gather(vbuf[vslot], [i1 + c])
                g2 = plsc.load_gather(vbuf[vslot], [i2 + c])
                acc = b0 * g0 + b1 * g1 + b2 * g2
                plsc.store_scatter(obuf[oslot], [lrv, cv + c], acc)

        return px_body

    ocopies = [None, None]
    vcopies = [None, None]
    vcopies[0] = pltpu.async_copy(verts_hbm.at[0], vbuf[0], vsem[0])
    for b in range(B):
        vslot = b % 2
        nxt = (b + 1) % 2
        if b + 1 < B:
            vcopies[nxt] = pltpu.async_copy(
                verts_hbm.at[b + 1], vbuf[nxt], vsem[nxt]
            )
        vcopies[vslot].wait()
        for chunk in range(NCH):
            oslot = (b * NCH + chunk) % 2
            if ocopies[oslot] is not None:
                ocopies[oslot].wait()
            plsc.parallel_loop(0, CHPX // L, unroll=4)(
                make_px_body(vslot, oslot, chunk)
            )
            ocopies[oslot] = pltpu.async_copy(
                obuf[oslot],
                out_hbm.at[b, pl.ds(row0 + chunk * (HPW // NCH), HPW // NCH)],
                osem[oslot],
            )
    for oc in ocopies:
        if oc is not None:
            oc.wait()


@functools.partial(jax.jit, static_argnames=())
def kernel(verts, bary_weights, v_index):
    idx3 = (v_index.reshape(P, C).astype(jnp.int32) * 3).T
    bary = bary_weights.reshape(P, C).T
    verts_flat = jnp.pad(
        verts.reshape(B, NV * C), ((0, 0), (0, NVP - NV * C))
    )

    sc = pl.kernel(
        _sc_body,
        mesh=plsc.VectorSubcoreMesh(core_axis_name="c", subcore_axis_name="s"),
        out_type=jax.ShapeDtypeStruct((B, H, W * C), jnp.float32),
        scratch_types=[
            pltpu.VMEM((C, PPW), jnp.int32),
            pltpu.VMEM((C, PPW), jnp.float32),
            pltpu.VMEM((NVP,), jnp.float32),
            pltpu.VMEM((NVP,), jnp.float32),
            pltpu.VMEM((HPW // NCH, W * C), jnp.float32),
            pltpu.VMEM((HPW // NCH, W * C), jnp.float32),
            pltpu.SemaphoreType.DMA,
            pltpu.SemaphoreType.DMA,
            pltpu.SemaphoreType.DMA,
            pltpu.SemaphoreType.DMA,
        ],
        compiler_params=pltpu.CompilerParams(needs_layout_passes=False),
    )
    return sc(verts_flat, idx3, bary).reshape(B, H, W, C)
